# baseline (device time: 81404 ns/iter reference)
import functools

import jax
import jax.numpy as jnp
from jax import lax
from jax.experimental import pallas as pl
from jax.experimental.pallas import tpu as pltpu

N_DEV = 16
N_LAYERS = 3
N_STAGES = 4


def kernel(x, Win0, Wout0, Win1, Wout1, Win2, Wout2):
    b, d_in = x.shape
    _, h_dim = Win0.shape

    def body(x_ref, win0_ref, wout0_ref, win1_ref, wout1_ref,
             win2_ref, wout2_ref, out_ref,
             acc_ref, recv_ref, send_sems, recv_sems):
        my = lax.axis_index("i")

        barrier_sem = pltpu.get_barrier_semaphore()
        for s in range(N_STAGES):
            partner = jnp.bitwise_xor(my, 1 << s)
            pl.semaphore_signal(
                barrier_sem, inc=1,
                device_id=(partner,), device_id_type=pl.DeviceIdType.MESH,
            )
        pl.semaphore_wait(barrier_sem, N_STAGES)

        win_refs = [win0_ref, win1_ref, win2_ref]
        wout_refs = [wout0_ref, wout1_ref, wout2_ref]

        x_val = x_ref[:, :]
        for l in range(N_LAYERS):
            acc_ref[:, :] = jnp.dot(
                x_val, win_refs[l][:, :], preferred_element_type=jnp.float32
            )
            for s in range(N_STAGES):
                partner = jnp.bitwise_xor(my, 1 << s)
                rdma = pltpu.make_async_remote_copy(
                    src_ref=acc_ref,
                    dst_ref=recv_ref.at[l, s],
                    send_sem=send_sems.at[l, s],
                    recv_sem=recv_sems.at[l, s],
                    device_id=(partner,),
                    device_id_type=pl.DeviceIdType.MESH,
                )
                rdma.start()
                rdma.wait()
                acc_ref[:, :] = acc_ref[:, :] + recv_ref[l, s]
            x_val = jnp.dot(
                jnp.maximum(acc_ref[:, :], 0.0),
                wout_refs[l][:, :],
                preferred_element_type=jnp.float32,
            )
        out_ref[:, :] = x_val

    return pl.pallas_call(
        body,
        out_shape=jax.ShapeDtypeStruct((b, d_in), jnp.float32),
        in_specs=[pl.BlockSpec(memory_space=pltpu.VMEM)] * 7,
        out_specs=pl.BlockSpec(memory_space=pltpu.VMEM),
        scratch_shapes=[
            pltpu.VMEM((b, h_dim), jnp.float32),
            pltpu.VMEM((N_LAYERS, N_STAGES, b, h_dim), jnp.float32),
            pltpu.SemaphoreType.DMA((N_LAYERS, N_STAGES)),
            pltpu.SemaphoreType.DMA((N_LAYERS, N_STAGES)),
        ],
        compiler_params=pltpu.CompilerParams(collective_id=0),
    )(x, Win0, Wout0, Win1, Wout1, Win2, Wout2)
